# baseline (speedup 1.0000x reference)
import functools

import jax
import jax.numpy as jnp
from jax import lax
from jax.experimental import pallas as pl
from jax.experimental.pallas import tpu as pltpu
from jax.experimental.pallas import tpu_sc as plsc

_NC = 2
_NS = 16
_NW = _NC * _NS

_BATCH = 1024
_SEQ = 200
_NTOK = 10
_DIM = 64
_OUT_S = _NTOK + _SEQ
_VOCAB = 1000000

_HALF = 1 << 19
_BLK_T = 4096
_NBT = _BATCH // 128
_SPW = 7


def _tc_pack(wte_t):
    nb = _HALF // _BLK_T

    def body(lo_ref, hi_ref, out_ref):
        out_ref[...] = jnp.concatenate(
            [lo_ref[...].T, hi_ref[...].T], axis=1)

    return pl.pallas_call(
        body,
        grid=(nb,),
        in_specs=[
            pl.BlockSpec((_DIM, _BLK_T), lambda k: (0, k)),
            pl.BlockSpec(
                (_DIM, _BLK_T),
                lambda k: (0, jnp.minimum(k + nb, _VOCAB // _BLK_T))),
        ],
        out_specs=pl.BlockSpec((_BLK_T, 2 * _DIM), lambda k: (k, 0)),
        out_shape=jax.ShapeDtypeStruct((_HALF, 2 * _DIM), jnp.float32),
    )(wte_t, wte_t)


def _make_sc_kernel():
    mesh = plsc.VectorSubcoreMesh(core_axis_name="c", subcore_axis_name="s")

    @functools.partial(
        pl.kernel,
        out_type=jax.ShapeDtypeStruct((_OUT_S * _DIM * _BATCH,), jnp.float32),
        mesh=mesh,
        scratch_types=[
            pltpu.VMEM((_BATCH,), jnp.int32),
            pltpu.VMEM((128,), jnp.int32),
            pltpu.VMEM((128,), jnp.int32),
            pltpu.VMEM((128, 128), jnp.float32),
            pltpu.VMEM((_DIM * 128,), jnp.float32),
            pltpu.SemaphoreType.DMA,
        ],
        compiler_params=pltpu.CompilerParams(needs_layout_passes=False),
    )
    def sc_gather(wtp_hbm, tok_hbm, lrn_hbm, out_hbm,
                  tokrow, idxs, baseb, rows, outb, sem):
        w = lax.axis_index("s") * _NC + lax.axis_index("c")

        lrn_base = w * (_NTOK * _DIM * _BATCH // _NW)
        @pl.loop(0, 5)
        def _lrn(c):
            off = lrn_base + c * 4096
            pltpu.sync_copy(lrn_hbm.at[pl.ds(off, 4096)],
                            outb.at[pl.ds(0, 4096)])
            pltpu.sync_copy(outb.at[pl.ds(0, 4096)],
                            out_hbm.at[pl.ds(off, 4096)])

        n_s = (_SEQ - 1 - w) // _NW + 1

        @pl.loop(0, n_s)
        def _souter(si):
            s_tok = si * _NW + w
            s = s_tok + _NTOK
            pltpu.sync_copy(tok_hbm.at[pl.ds(s_tok * _BATCH, _BATCH)], tokrow)

            @pl.loop(0, _NBT)
            def _bt(bt):
                @pl.loop(0, 8)
                def _prep(g):
                    tok = tokrow[pl.ds(bt * 128 + g * 16, 16)]
                    idxs[pl.ds(g * 16, 16)] = tok & (_HALF - 1)
                    baseb[pl.ds(g * 16, 16)] = (tok >> 19) * _DIM

                pltpu.async_copy(wtp_hbm.at[idxs], rows, sem).wait()

                @pl.loop(0, _DIM)
                def _d(d):
                    for g in range(8):
                        rowv = g * 16 + lax.iota(jnp.int32, 16)
                        colv = baseb[pl.ds(g * 16, 16)] + d
                        v = plsc.load_gather(rows, [rowv, colv])
                        outb[pl.ds(d * 128 + g * 16, 16)] = v

                @pl.loop(0, 8)
                def _out(dt):
                    dst = ((s * 8 + dt) * _NBT + bt) * 1024
                    pltpu.sync_copy(outb.at[pl.ds(dt * 1024, 1024)],
                                    out_hbm.at[pl.ds(dst, 1024)])

    return sc_gather


_SC_GATHER = _make_sc_kernel()


def kernel(tokens, wte_weight, learned_embedding):
    wtp = _tc_pack(wte_weight.T)
    tok_flat = tokens.astype(jnp.int32).T.reshape(-1)
    lrn = jnp.broadcast_to(
        learned_embedding.reshape(_NTOK, 8, 1, 8, 1),
        (_NTOK, 8, _NBT, 8, 128)).reshape(-1)
    out_flat = _SC_GATHER(wtp, tok_flat, lrn)
    return (out_flat.reshape(_OUT_S, 8, _NBT, 8, 128)
            .transpose(2, 4, 0, 1, 3)
            .reshape(_BATCH, _OUT_S, _DIM))


# speedup vs baseline: 1.1105x; 1.1105x over previous
import functools

import jax
import jax.numpy as jnp
from jax import lax
from jax.experimental import pallas as pl
from jax.experimental.pallas import tpu as pltpu
from jax.experimental.pallas import tpu_sc as plsc

_NC = 2
_NS = 16
_NW = _NC * _NS

_BATCH = 1024
_SEQ = 200
_NTOK = 10
_DIM = 64
_OUT_S = _NTOK + _SEQ
_VOCAB = 1000000

_HALF = 1 << 19
_BLK_T = 4096
_NBT = _BATCH // 128
_SPW = 7


def _tc_pack(wte_t):
    nb = _HALF // _BLK_T

    def body(lo_ref, hi_ref, out_ref):
        out_ref[...] = jnp.concatenate(
            [lo_ref[...].T, hi_ref[...].T], axis=1)

    return pl.pallas_call(
        body,
        grid=(nb,),
        in_specs=[
            pl.BlockSpec((_DIM, _BLK_T), lambda k: (0, k)),
            pl.BlockSpec(
                (_DIM, _BLK_T),
                lambda k: (0, jnp.minimum(k + nb, _VOCAB // _BLK_T))),
        ],
        out_specs=pl.BlockSpec((_BLK_T, 2 * _DIM), lambda k: (k, 0)),
        out_shape=jax.ShapeDtypeStruct((_HALF, 2 * _DIM), jnp.float32),
    )(wte_t, wte_t)


def _make_sc_kernel():
    mesh = plsc.VectorSubcoreMesh(core_axis_name="c", subcore_axis_name="s")

    @functools.partial(
        pl.kernel,
        out_type=jax.ShapeDtypeStruct((_OUT_S, 8, _NBT, 8, 128), jnp.float32),
        mesh=mesh,
        scratch_types=[
            pltpu.VMEM((_BATCH,), jnp.int32),
            pltpu.VMEM((_BATCH,), jnp.int32),
            pltpu.VMEM((_BATCH,), jnp.int32),
            pltpu.VMEM((2, 128, 128), jnp.float32),
            pltpu.VMEM((2, 8, 8, 128), jnp.float32),
            pltpu.SemaphoreType.DMA,
            pltpu.SemaphoreType.DMA,
            pltpu.SemaphoreType.DMA,
            pltpu.SemaphoreType.DMA,
        ],
        compiler_params=pltpu.CompilerParams(needs_layout_passes=False),
    )
    def sc_gather(wtp_hbm, tok_hbm, lrn_hbm, out_hbm,
                  tokrow, idxa, basea, rows, outb,
                  gsem0, gsem1, ssem0, ssem1):
        w = lax.axis_index("s") * _NC + lax.axis_index("c")
        gsems = (gsem0, gsem1)
        ssems = (ssem0, ssem1)

        @pl.loop(0, 20)
        def _lrn(c):
            row = w * 20 + c
            s0 = row // 64
            dt0 = (row % 64) // 8
            bt0 = row % 8
            pltpu.sync_copy(lrn_hbm.at[s0, dt0, bt0], rows.at[0, pl.ds(0, 8)])
            pltpu.sync_copy(rows.at[0, pl.ds(0, 8)], out_hbm.at[s0, dt0, bt0])

        n_s = (_SEQ - 1 - w) // _NW + 1

        def transpose_block(bt, slot):
            @pl.loop(0, _DIM, unroll=4)
            def _d(d):
                dt = d >> 3
                ds = d & 7
                for g in range(8):
                    rowv = g * 16 + lax.iota(jnp.int32, 16)
                    colv = basea[pl.ds(bt * 128 + g * 16, 16)] + d
                    v = plsc.load_gather(rows.at[slot], [rowv, colv])
                    outb[slot, dt, ds, pl.ds(g * 16, 16)] = v

        @pl.loop(0, n_s)
        def _souter(si):
            s_tok = si * _NW + w
            s = s_tok + _NTOK
            pltpu.sync_copy(tok_hbm.at[pl.ds(s_tok * _BATCH, _BATCH)], tokrow)

            @pl.loop(0, 64, unroll=4)
            def _prep(g):
                tok = tokrow[pl.ds(g * 16, 16)]
                idxa[pl.ds(g * 16, 16)] = tok & (_HALF - 1)
                basea[pl.ds(g * 16, 16)] = (tok >> 19) * _DIM

            def issue_gather(bt, slot):
                return pltpu.async_copy(
                    wtp_hbm.at[idxa.at[pl.ds(bt * 128, 128)]],
                    rows.at[slot], gsems[slot])

            g_desc = [issue_gather(0, 0), None]
            s_desc = [None, None]
            for bt in range(_NBT):
                slot = bt & 1
                if bt + 1 < _NBT:
                    g_desc[1 - slot] = issue_gather(bt + 1, 1 - slot)
                g_desc[slot].wait()
                if s_desc[slot] is not None:
                    s_desc[slot].wait()
                transpose_block(bt, slot)
                s_desc[slot] = pltpu.async_copy(
                    outb.at[slot], out_hbm.at[s, :, bt], ssems[slot])
            s_desc[0].wait()
            s_desc[1].wait()

    return sc_gather


_SC_GATHER = _make_sc_kernel()


def kernel(tokens, wte_weight, learned_embedding):
    wtp = _tc_pack(wte_weight.T)
    tok_flat = tokens.astype(jnp.int32).T.reshape(-1)
    lrn = jnp.broadcast_to(
        learned_embedding.reshape(_NTOK, 8, 1, 8, 1),
        (_NTOK, 8, _NBT, 8, 128))
    out5 = _SC_GATHER(wtp, tok_flat, lrn)
    return (out5.transpose(2, 4, 0, 1, 3)
            .reshape(_BATCH, _OUT_S, _DIM))
